# baseline (speedup 1.0000x reference)
import functools

import jax
import jax.numpy as jnp
from jax import lax
from jax.experimental import pallas as pl
from jax.experimental.pallas import tpu as pltpu
from jax.experimental.pallas import tpu_sc as plsc

B = 1024
L = 200
H = 128
N = B * L
NW = 32
PER_W = N // NW
C = 128
NCHUNK = PER_W // C
NCOL = H // 16


def _sc_body(x_hbm, xseg_hbm, tok_hbm, pos_hbm, seg_hbm, out_hbm,
             xi, si, acc, tmp_pos, tmp_seg, sem_t, sem_p, sem_s):
    wid = lax.axis_index("s") * 2 + lax.axis_index("c")
    base = wid * PER_W

    def chunk_body(ci, carry):
        off = base + ci * C
        pltpu.sync_copy(x_hbm.at[pl.ds(off, C)], xi)
        pltpu.sync_copy(xseg_hbm.at[pl.ds(off, C)], si)
        cp_t = pltpu.async_copy(tok_hbm.at[xi], acc, sem_t)
        cp_p = pltpu.async_copy(pos_hbm.at[xi], tmp_pos, sem_p)
        cp_s = pltpu.async_copy(seg_hbm.at[si], tmp_seg, sem_s)
        cp_t.wait()
        cp_p.wait()

        def add_pos(r, c2):
            for j in range(NCOL):
                v = tmp_pos[r, pl.ds(j * 16, 16)]
                plsc.addupdate(acc.at[r, pl.ds(j * 16, 16)], v)
            return c2

        lax.fori_loop(0, C, add_pos, 0)
        cp_s.wait()

        def add_seg(r, c2):
            for j in range(NCOL):
                v = tmp_seg[r, pl.ds(j * 16, 16)]
                plsc.addupdate(acc.at[r, pl.ds(j * 16, 16)], v)
            return c2

        lax.fori_loop(0, C, add_seg, 0)
        pltpu.sync_copy(acc, out_hbm.at[pl.ds(off, C)])
        return carry

    lax.fori_loop(0, NCHUNK, chunk_body, 0)


@jax.jit
def _run(x_flat, xseg_flat, token_table, pos_table, seg_table):
    mesh = plsc.VectorSubcoreMesh(core_axis_name="c", subcore_axis_name="s")
    call = pl.kernel(
        _sc_body,
        out_type=jax.ShapeDtypeStruct((N, H), jnp.float32),
        mesh=mesh,
        scratch_types=[
            pltpu.VMEM((C,), jnp.int32),
            pltpu.VMEM((C,), jnp.int32),
            pltpu.VMEM((C, H), jnp.float32),
            pltpu.VMEM((C, H), jnp.float32),
            pltpu.VMEM((C, H), jnp.float32),
            pltpu.SemaphoreType.DMA,
            pltpu.SemaphoreType.DMA,
            pltpu.SemaphoreType.DMA,
        ],
    )
    return call(x_flat, xseg_flat, token_table, pos_table, seg_table)


def kernel(x, x_seg, token_table, pos_table, seg_table):
    x_flat = x.reshape(N)
    xseg_flat = x_seg.reshape(N)
    out = _run(x_flat, xseg_flat, token_table, pos_table, seg_table)
    return out.reshape(B, L, H)


# speedup vs baseline: 14.0421x; 14.0421x over previous
import functools

import jax
import jax.numpy as jnp
from jax import lax
from jax.experimental import pallas as pl
from jax.experimental.pallas import tpu as pltpu
from jax.experimental.pallas import tpu_sc as plsc

B = 1024
L = 200
H = 128
POS_ROWS = 513
SEG_ROWS = 3
N = B * L
NW = 32
PER_W = N // NW
C = 128
NCHUNK = PER_W // C
NCOL = H // 16


def _comb_tc_body(pos_ref, seg_ref, out_ref):
    out_ref[...] = seg_ref[...][:, None, :] + pos_ref[...][None, :, :]


def _sc_body(x_hbm, xseg_hbm, tok_hbm, comb_hbm, out_hbm,
             xi, si, ci, tok0, tok1, cb0, cb1, st0, st1,
             sg0, sg1, so0, so1):
    wid = lax.axis_index("s") * 2 + lax.axis_index("c")
    base = wid * PER_W
    pltpu.sync_copy(x_hbm.at[wid], xi)
    pltpu.sync_copy(xseg_hbm.at[wid], si)

    def mkidx(r, carry):
        for j in range(NCOL):
            sl = (r, pl.ds(j * 16, 16))
            ci[sl] = si[sl] * POS_ROWS + xi[sl]
        return carry

    lax.fori_loop(0, NCHUNK, mkidx, 0)

    toks = (tok0, tok1)
    cbs = (cb0, cb1)
    stages = (st0, st1)
    sgs = (sg0, sg1)
    sos = (so0, so1)

    def issue(i, b):
        pltpu.async_copy(tok_hbm.at[xi.at[i]], toks[b], sgs[b])
        pltpu.async_copy(comb_hbm.at[ci.at[i]], cbs[b], sgs[b])

    def wait_gathers(b):
        pltpu.make_async_copy(tok_hbm.at[xi.at[0]], toks[b], sgs[b]).wait()
        pltpu.make_async_copy(comb_hbm.at[ci.at[0]], cbs[b], sgs[b]).wait()

    def wait_out(b):
        pltpu.make_async_copy(
            stages[b], out_hbm.at[pl.ds(base, C)], sos[b]).wait()

    def add_rows(b):
        tok, cb, st = toks[b], cbs[b], stages[b]

        def row(r, carry):
            for j in range(NCOL):
                sl = (r, pl.ds(j * 16, 16))
                st[sl] = tok[sl] + cb[sl]
            return carry

        lax.fori_loop(0, C, row, 0)

    issue(0, 0)
    issue(1, 1)

    def step(k, carry):
        for b in range(2):
            i = 2 * k + b
            wait_gathers(b)

            @pl.when(k > 0)
            def _():
                wait_out(b)

            add_rows(b)

            @pl.when(i + 2 < NCHUNK)
            def _():
                issue(i + 2, b)

            pltpu.async_copy(
                stages[b], out_hbm.at[pl.ds(base + i * C, C)], sos[b])
        return carry

    lax.fori_loop(0, NCHUNK // 2, step, 0)
    wait_out(0)
    wait_out(1)


@jax.jit
def _run(x3d, xseg3d, token_table, pos_table, seg_table):
    comb = pl.pallas_call(
        _comb_tc_body,
        out_shape=jax.ShapeDtypeStruct((SEG_ROWS, POS_ROWS, H), jnp.float32),
    )(pos_table, seg_table)
    comb = comb.reshape(SEG_ROWS * POS_ROWS, H)

    mesh = plsc.VectorSubcoreMesh(core_axis_name="c", subcore_axis_name="s")
    call = pl.kernel(
        _sc_body,
        out_type=jax.ShapeDtypeStruct((N, H), jnp.float32),
        mesh=mesh,
        scratch_types=[
            pltpu.VMEM((NCHUNK, C), jnp.int32),
            pltpu.VMEM((NCHUNK, C), jnp.int32),
            pltpu.VMEM((NCHUNK, C), jnp.int32),
            pltpu.VMEM((C, H), jnp.float32),
            pltpu.VMEM((C, H), jnp.float32),
            pltpu.VMEM((C, H), jnp.float32),
            pltpu.VMEM((C, H), jnp.float32),
            pltpu.VMEM((C, H), jnp.float32),
            pltpu.VMEM((C, H), jnp.float32),
            pltpu.SemaphoreType.DMA,
            pltpu.SemaphoreType.DMA,
            pltpu.SemaphoreType.DMA,
            pltpu.SemaphoreType.DMA,
        ],
    )
    return call(x3d, xseg3d, token_table, comb)


def kernel(x, x_seg, token_table, pos_table, seg_table):
    x3d = x.reshape(NW, NCHUNK, C)
    xseg3d = x_seg.reshape(NW, NCHUNK, C)
    out = _run(x3d, xseg3d, token_table, pos_table, seg_table)
    return out.reshape(B, L, H)


# speedup vs baseline: 23.2123x; 1.6531x over previous
import functools

import jax
import jax.numpy as jnp
from jax import lax
from jax.experimental import pallas as pl
from jax.experimental.pallas import tpu as pltpu
from jax.experimental.pallas import tpu_sc as plsc

B = 1024
L = 200
H = 128
POS_ROWS = 513
SEG_ROWS = 3
N = B * L
NW = 32
PER_W = N // NW
C = 128
NCHUNK = PER_W // C
NBUF = 4
NCOL = H // 16


def _fused_tc_body(tok_ref, pos_ref, seg_ref, out_ref):
    tp = tok_ref[...] + pos_ref[...]
    out_ref[...] = tp[None, :, :] + seg_ref[...][:, None, :]


def _sc_body(x_hbm, xseg_hbm, fused_hbm, out_hbm,
             xi, si, b0, b1, b2, b3, sg0, sg1, sg2, sg3,
             so0, so1, so2, so3):
    wid = lax.axis_index("s") * 2 + lax.axis_index("c")
    base = wid * PER_W
    pltpu.sync_copy(x_hbm.at[wid], xi)
    pltpu.sync_copy(xseg_hbm.at[wid], si)

    def mkidx(r, carry):
        for j in range(NCOL):
            sl = (r, pl.ds(j * 16, 16))
            si[sl] = si[sl] * POS_ROWS + xi[sl]
        return carry

    lax.fori_loop(0, NCHUNK, mkidx, 0)

    bufs = (b0, b1, b2, b3)
    sgs = (sg0, sg1, sg2, sg3)
    sos = (so0, so1, so2, so3)

    def issue(i, b):
        pltpu.async_copy(fused_hbm.at[si.at[i]], bufs[b], sgs[b])

    def wait_gather(b):
        pltpu.make_async_copy(fused_hbm.at[si.at[0]], bufs[b], sgs[b]).wait()

    def wait_out(b):
        pltpu.make_async_copy(
            bufs[b], out_hbm.at[pl.ds(base, C)], sos[b]).wait()

    for b in range(NBUF - 1):
        issue(b, b)

    def step(k, carry):
        for b in range(NBUF):
            i = NBUF * k + b
            wait_gather(b)
            pltpu.async_copy(
                bufs[b], out_hbm.at[pl.ds(base + i * C, C)], sos[b])
            nxt = (b + NBUF - 1) % NBUF

            @pl.when(NBUF * k + b + NBUF - 1 < NCHUNK)
            def _():
                @pl.when(k + b > 0)
                def _():
                    wait_out(nxt)

                issue(i + NBUF - 1, nxt)
        return carry

    lax.fori_loop(0, NCHUNK // NBUF, step, 0)
    for i in range(NBUF * (NCHUNK // NBUF), NCHUNK):
        b = i % NBUF
        wait_gather(b)
        pltpu.async_copy(
            bufs[b], out_hbm.at[pl.ds(base + i * C, C)], sos[b])
    for i in range(NCHUNK - NBUF, NCHUNK):
        wait_out(i % NBUF)


@jax.jit
def _run(x3d, xseg3d, tok513, pos_table, seg_table):
    fused = pl.pallas_call(
        _fused_tc_body,
        out_shape=jax.ShapeDtypeStruct((SEG_ROWS, POS_ROWS, H), jnp.float32),
    )(tok513, pos_table, seg_table)
    fused = fused.reshape(SEG_ROWS * POS_ROWS, H)

    mesh = plsc.VectorSubcoreMesh(core_axis_name="c", subcore_axis_name="s")
    call = pl.kernel(
        _sc_body,
        out_type=jax.ShapeDtypeStruct((N, H), jnp.float32),
        mesh=mesh,
        scratch_types=[
            pltpu.VMEM((NCHUNK, C), jnp.int32),
            pltpu.VMEM((NCHUNK, C), jnp.int32),
            pltpu.VMEM((C, H), jnp.float32),
            pltpu.VMEM((C, H), jnp.float32),
            pltpu.VMEM((C, H), jnp.float32),
            pltpu.VMEM((C, H), jnp.float32),
            pltpu.SemaphoreType.DMA,
            pltpu.SemaphoreType.DMA,
            pltpu.SemaphoreType.DMA,
            pltpu.SemaphoreType.DMA,
            pltpu.SemaphoreType.DMA,
            pltpu.SemaphoreType.DMA,
            pltpu.SemaphoreType.DMA,
            pltpu.SemaphoreType.DMA,
        ],
    )
    return call(x3d, xseg3d, fused)


def kernel(x, x_seg, token_table, pos_table, seg_table):
    x3d = x.reshape(NW, NCHUNK, C)
    xseg3d = x_seg.reshape(NW, NCHUNK, C)
    out = _run(x3d, xseg3d, token_table[:POS_ROWS], pos_table, seg_table)
    return out.reshape(B, L, H)
